# baseline (speedup 1.0000x reference)
import functools

import jax
import jax.numpy as jnp
from jax import lax
from jax.experimental import pallas as pl
from jax.experimental.pallas import tpu as pltpu
from jax.experimental.pallas import tpu_sc as plsc

_NC = 2
_NS = 16
_NW = _NC * _NS
_CH = 80
_LANES = 128


@functools.lru_cache(maxsize=None)
def _make_agg(n_pad: int, n_edges: int, with_counts: bool):
    ept = n_edges // _NW
    nch = ept // _CH
    ncr = n_pad // _LANES
    n_acc = n_pad + (_LANES if with_counts else 0)
    rpt = n_acc // _NS
    mesh = plsc.VectorSubcoreMesh(core_axis_name="c", subcore_axis_name="s")

    assert nch % 2 == 1

    scratch = [
        pltpu.VMEM((_CH,), jnp.int32),
        pltpu.VMEM((_CH,), jnp.int32),
        pltpu.VMEM((nch, _CH), jnp.int32),
        pltpu.VMEM((_CH, _LANES), jnp.float32),
        pltpu.VMEM((_CH, _LANES), jnp.float32),
        pltpu.VMEM_SHARED((n_acc, _LANES), jnp.float32),
        pltpu.SemaphoreType.DMA,
        pltpu.SemaphoreType.DMA,
        pltpu.SemaphoreType.DMA,
        pltpu.SemaphoreType.DMA,
        pltpu.SemaphoreType.DMA,
        pltpu.SemaphoreType.DMA,
    ]
    if with_counts:
        scratch += [
            pltpu.VMEM((ncr, _LANES), jnp.float32),
            pltpu.VMEM((ncr,), jnp.int32),
        ]

    @functools.partial(
        pl.kernel,
        mesh=mesh,
        compiler_params=pltpu.CompilerParams(needs_layout_passes=False),
        out_type=jax.ShapeDtypeStruct((_NC, n_acc, _LANES), jnp.float32),
        scratch_types=scratch,
    )
    def agg(x_hbm, src_hbm, dst_hbm, zeros_hbm, out_hbm,
            srcb_a, srcb_b, dst_v, rows_a, rows_b, acc_sh,
            gsem_a, gsem_b, ssem_a, ssem_b, isem_a, isem_b, *extra):
        c = lax.axis_index("c")
        s = lax.axis_index("s")
        wid = s * _NC + c
        pltpu.sync_copy(zeros_hbm, acc_sh.at[pl.ds(s * rpt, rpt)])
        pltpu.sync_copy(dst_hbm.at[wid], dst_v)
        if with_counts:
            hist_v, cidx_v = extra
            pltpu.sync_copy(zeros_hbm.at[pl.ds(0, ncr)], hist_v)
            for k in range(ncr // 16):
                cidx_v[pl.ds(16 * k, 16)] = (
                    n_pad + 16 * k + lax.iota(jnp.int32, 16))
        plsc.subcore_barrier()

        ones16 = jnp.ones((16,), jnp.float32)
        buf_a = (srcb_a, rows_a, gsem_a, ssem_a, isem_a)
        buf_b = (srcb_b, rows_b, gsem_b, ssem_b, isem_b)

        def hist_add(j):
            if with_counts:
                for k in range(_CH // 16):
                    d16 = dst_v[j, pl.ds(16 * k, 16)]
                    plsc.addupdate_scatter(
                        hist_v,
                        [jnp.right_shift(d16, 7), jnp.bitwise_and(d16, 127)],
                        ones16)

        def handle(j, me, ot):
            srcb_me, rows_me, gsem_me, ssem_me, isem_me = me
            srcb_ot, rows_ot, gsem_ot, ssem_ot, isem_ot = ot

            @pl.when(j > 0)
            def _():
                pltpu.make_async_copy(
                    rows_ot, acc_sh.at[dst_v.at[j - 1]], ssem_ot).wait()

            pltpu.make_async_copy(
                src_hbm.at[wid, j + 1], srcb_ot, isem_ot).wait()
            pltpu.async_copy(x_hbm.at[srcb_ot], rows_ot, gsem_ot)
            hist_add(j)
            pltpu.make_async_copy(x_hbm.at[srcb_me], rows_me, gsem_me).wait()

            @pl.when(j + 2 < nch)
            def _():
                pltpu.async_copy(src_hbm.at[wid, j + 2], srcb_me, isem_me)

            pltpu.async_copy(rows_me, acc_sh.at[dst_v.at[j]], ssem_me,
                             add=True)

        pltpu.sync_copy(src_hbm.at[wid, 0], srcb_a)
        pltpu.async_copy(x_hbm.at[srcb_a], rows_a, gsem_a)
        pltpu.async_copy(src_hbm.at[wid, 1], srcb_b, isem_b)

        def pair(i, carry):
            j0 = 2 * i
            handle(j0, buf_a, buf_b)
            handle(j0 + 1, buf_b, buf_a)
            return carry

        lax.fori_loop(0, (nch - 1) // 2, pair, 0)

        jt = nch - 1
        pltpu.make_async_copy(
            rows_b, acc_sh.at[dst_v.at[jt - 1]], ssem_b).wait()
        pltpu.make_async_copy(x_hbm.at[srcb_a], rows_a, gsem_a).wait()
        pltpu.sync_copy(rows_a, acc_sh.at[dst_v.at[jt]], add=True)
        hist_add(jt)

        if with_counts:
            pltpu.sync_copy(hist_v, acc_sh.at[cidx_v], add=True)
        plsc.subcore_barrier()
        pltpu.sync_copy(acc_sh.at[pl.ds(s * rpt, rpt)],
                        out_hbm.at[c, pl.ds(s * rpt, rpt)])

    return agg


def _right_block(x_ref, wr_ref, b_ref, xr_ref):
    xr_ref[...] = (jnp.dot(x_ref[...], wr_ref[...],
                           preferred_element_type=jnp.float32) + b_ref[...])


def _layer1_block(ps_ref, pc_ref, xr_ref, wl_ref, h_ref, invc_ref):
    blk = ps_ref.shape[1]
    nr = blk // _LANES
    cnt = pc_ref[0] + pc_ref[1]
    invc = 1.0 / jnp.maximum(cnt, 1.0)
    summed = (ps_ref[0] + ps_ref[1]).reshape(nr, _LANES, _LANES)
    mean = (summed * invc[:, :, None]).reshape(blk, _LANES)
    h = (jnp.dot(mean, wl_ref[...], preferred_element_type=jnp.float32)
         + xr_ref[...])
    h_ref[...] = jnp.maximum(h, 0.0)
    invc_ref[...] = invc


def _layer2_block(ps_ref, invc_ref, hr_ref, wl_ref, o_ref):
    blk = ps_ref.shape[1]
    nr = blk // _LANES
    summed = (ps_ref[0] + ps_ref[1]).reshape(nr, _LANES, _LANES)
    mean = (summed * invc_ref[...][:, :, None]).reshape(blk, _LANES)
    o = (jnp.dot(mean, wl_ref[...], preferred_element_type=jnp.float32)
         + hr_ref[...])
    nrm = jnp.sqrt(jnp.sum(o * o, axis=1, keepdims=True))
    o = o / jnp.maximum(nrm, 1e-12)
    o_ref[...] = jnp.maximum(o, 0.0)


def _tc_right(x, wr, b, n_pad, blk):
    return pl.pallas_call(
        _right_block,
        grid=(n_pad // blk,),
        in_specs=[
            pl.BlockSpec((blk, _LANES), lambda i: (i, 0)),
            pl.BlockSpec((_LANES, _LANES), lambda i: (0, 0)),
            pl.BlockSpec((1, _LANES), lambda i: (0, 0)),
        ],
        out_specs=pl.BlockSpec((blk, _LANES), lambda i: (i, 0)),
        out_shape=jax.ShapeDtypeStruct((n_pad, _LANES), jnp.float32),
    )(x, wr, b)


def _tc_layer1(p, xr, wl, n_pad, blk):
    nr = blk // _LANES
    return pl.pallas_call(
        _layer1_block,
        grid=(n_pad // blk,),
        in_specs=[
            pl.BlockSpec((2, blk, _LANES), lambda i: (0, i, 0)),
            pl.BlockSpec((2, nr, _LANES),
                         lambda i: (0, n_pad // nr + i, 0)),
            pl.BlockSpec((blk, _LANES), lambda i: (i, 0)),
            pl.BlockSpec((_LANES, _LANES), lambda i: (0, 0)),
        ],
        out_specs=[
            pl.BlockSpec((blk, _LANES), lambda i: (i, 0)),
            pl.BlockSpec((nr, _LANES), lambda i: (i, 0)),
        ],
        out_shape=[
            jax.ShapeDtypeStruct((n_pad, _LANES), jnp.float32),
            jax.ShapeDtypeStruct((n_pad // _LANES, _LANES), jnp.float32),
        ],
    )(p, p, xr, wl)


def _tc_layer2(p2, invc, hr, wl, n_pad, blk):
    nr = blk // _LANES
    return pl.pallas_call(
        _layer2_block,
        grid=(n_pad // blk,),
        in_specs=[
            pl.BlockSpec((2, blk, _LANES), lambda i: (0, i, 0)),
            pl.BlockSpec((nr, _LANES), lambda i: (i, 0)),
            pl.BlockSpec((blk, _LANES), lambda i: (i, 0)),
            pl.BlockSpec((_LANES, _LANES), lambda i: (0, 0)),
        ],
        out_specs=pl.BlockSpec((blk, _LANES), lambda i: (i, 0)),
        out_shape=jax.ShapeDtypeStruct((n_pad, _LANES), jnp.float32),
    )(p2, invc, hr, wl)


def kernel(matrix_nodes_features, edge_index, W1l, b1, W1r, W2l, b2, W2r):
    x = matrix_nodes_features
    n, d = x.shape
    e = edge_index.shape[1]
    n_pad = -(-n // 2048) * 2048
    blk = 2048

    src = edge_index[0].astype(jnp.int32)
    dst = edge_index[1].astype(jnp.int32)
    ept = e // _NW
    nch = ept // _CH
    src3 = src.reshape(_NW, nch, _CH)
    dst3 = dst.reshape(_NW, nch, _CH)

    x_pad = jnp.zeros((n_pad, d), jnp.float32).at[:n].set(x)
    n_acc1 = n_pad + _LANES
    z_cnt = jnp.zeros((n_acc1 // _NS, _LANES), jnp.float32)
    z_pln = jnp.zeros((n_pad // _NS, _LANES), jnp.float32)

    agg1 = _make_agg(n_pad, e, True)
    agg2 = _make_agg(n_pad, e, False)

    p1 = agg1(x_pad, src3, dst3, z_cnt)
    xr = _tc_right(x_pad, W1r, b1.reshape(1, -1), n_pad, blk)
    h, invc = _tc_layer1(p1, xr, W1l, n_pad, blk)
    p2 = agg2(h, src3, dst3, z_pln)
    hr = _tc_right(h, W2r, b2.reshape(1, -1), n_pad, blk)
    out = _tc_layer2(p2, invc, hr, W2l, n_pad, blk)
    return out[:n]


# speedup vs baseline: 1.0343x; 1.0343x over previous
import functools

import jax
import jax.numpy as jnp
from jax import lax
from jax.experimental import pallas as pl
from jax.experimental.pallas import tpu as pltpu
from jax.experimental.pallas import tpu_sc as plsc

_NC = 2
_NS = 16
_NW = _NC * _NS
_CH = 80
_LANES = 128


@functools.lru_cache(maxsize=None)
def _make_agg(n_pad: int, n_edges: int, with_counts: bool):
    ept = n_edges // _NW
    nch = ept // _CH
    ncr = n_pad // _LANES
    n_acc = n_pad + (_LANES if with_counts else 0)
    rpt = n_acc // _NS
    mesh = plsc.VectorSubcoreMesh(core_axis_name="c", subcore_axis_name="s")

    assert nch % 2 == 1

    scratch = [
        pltpu.VMEM((_CH,), jnp.int32),
        pltpu.VMEM((_CH,), jnp.int32),
        pltpu.VMEM((nch, _CH), jnp.int32),
        pltpu.VMEM((_CH, _LANES), jnp.float32),
        pltpu.VMEM((_CH, _LANES), jnp.float32),
        pltpu.VMEM_SHARED((n_acc, _LANES), jnp.float32),
        pltpu.SemaphoreType.DMA,
        pltpu.SemaphoreType.DMA,
        pltpu.SemaphoreType.DMA,
        pltpu.SemaphoreType.DMA,
        pltpu.SemaphoreType.DMA,
        pltpu.SemaphoreType.DMA,
    ]
    if with_counts:
        scratch += [
            pltpu.VMEM((ncr, _LANES), jnp.float32),
            pltpu.VMEM((ncr,), jnp.int32),
        ]

    @functools.partial(
        pl.kernel,
        mesh=mesh,
        compiler_params=pltpu.CompilerParams(needs_layout_passes=False),
        out_type=jax.ShapeDtypeStruct((_NC, n_acc, _LANES), jnp.float32),
        scratch_types=scratch,
    )
    def agg(x_hbm, src_hbm, dst_hbm, zeros_hbm, out_hbm,
            srcb_a, srcb_b, dst_v, rows_a, rows_b, acc_sh,
            gsem_a, gsem_b, ssem_a, ssem_b, isem_a, isem_b, *extra):
        c = lax.axis_index("c")
        s = lax.axis_index("s")
        wid = s * _NC + c
        pltpu.sync_copy(zeros_hbm, acc_sh.at[pl.ds(s * rpt, rpt)])
        pltpu.sync_copy(dst_hbm.at[wid], dst_v)
        if with_counts:
            hist_v, cidx_v = extra
            pltpu.sync_copy(zeros_hbm.at[pl.ds(0, ncr)], hist_v)
            for k in range(ncr // 16):
                cidx_v[pl.ds(16 * k, 16)] = (
                    n_pad + 16 * k + lax.iota(jnp.int32, 16))
        plsc.subcore_barrier()

        ones16 = jnp.ones((16,), jnp.float32)
        buf_a = (srcb_a, rows_a, gsem_a, ssem_a, isem_a)
        buf_b = (srcb_b, rows_b, gsem_b, ssem_b, isem_b)

        def hist_add(j):
            if with_counts:
                for k in range(_CH // 16):
                    d16 = dst_v[j, pl.ds(16 * k, 16)]
                    plsc.addupdate_scatter(
                        hist_v,
                        [jnp.right_shift(d16, 7), jnp.bitwise_and(d16, 127)],
                        ones16)

        def handle(j, me, ot):
            srcb_me, rows_me, gsem_me, ssem_me, isem_me = me
            srcb_ot, rows_ot, gsem_ot, ssem_ot, isem_ot = ot

            @pl.when(j > 0)
            def _():
                pltpu.make_async_copy(
                    rows_ot, acc_sh.at[dst_v.at[j - 1]], ssem_ot).wait()

            pltpu.make_async_copy(
                src_hbm.at[wid, j + 1], srcb_ot, isem_ot).wait()
            pltpu.async_copy(x_hbm.at[srcb_ot], rows_ot, gsem_ot)
            hist_add(j)
            pltpu.make_async_copy(x_hbm.at[srcb_me], rows_me, gsem_me).wait()

            @pl.when(j + 2 < nch)
            def _():
                pltpu.async_copy(src_hbm.at[wid, j + 2], srcb_me, isem_me)

            pltpu.async_copy(rows_me, acc_sh.at[dst_v.at[j]], ssem_me,
                             add=True)

        pltpu.sync_copy(src_hbm.at[wid, 0], srcb_a)
        pltpu.async_copy(x_hbm.at[srcb_a], rows_a, gsem_a)
        pltpu.async_copy(src_hbm.at[wid, 1], srcb_b, isem_b)

        def pair(i, carry):
            j0 = 2 * i
            handle(j0, buf_a, buf_b)
            handle(j0 + 1, buf_b, buf_a)
            return carry

        lax.fori_loop(0, (nch - 1) // 2, pair, 0)

        jt = nch - 1
        pltpu.make_async_copy(
            rows_b, acc_sh.at[dst_v.at[jt - 1]], ssem_b).wait()
        pltpu.make_async_copy(x_hbm.at[srcb_a], rows_a, gsem_a).wait()
        pltpu.sync_copy(rows_a, acc_sh.at[dst_v.at[jt]], add=True)
        hist_add(jt)

        if with_counts:
            pltpu.sync_copy(hist_v, acc_sh.at[cidx_v], add=True)
        plsc.subcore_barrier()
        pltpu.sync_copy(acc_sh.at[pl.ds(s * rpt, rpt)],
                        out_hbm.at[c, pl.ds(s * rpt, rpt)])

    return agg


def _layer1_block(ps_ref, pc_ref, x_ref, wl_ref, b_ref, wr_ref,
                  h_ref, invc_ref):
    blk = ps_ref.shape[1]
    nr = blk // _LANES
    cnt = pc_ref[0] + pc_ref[1]
    invc = 1.0 / jnp.maximum(cnt, 1.0)
    summed = (ps_ref[0] + ps_ref[1]).reshape(nr, _LANES, _LANES)
    mean = (summed * invc[:, :, None]).reshape(blk, _LANES)
    h = (jnp.dot(mean, wl_ref[...], preferred_element_type=jnp.float32)
         + b_ref[...]
         + jnp.dot(x_ref[...], wr_ref[...], preferred_element_type=jnp.float32))
    h_ref[...] = jnp.maximum(h, 0.0)
    invc_ref[...] = invc


def _layer2_block(ps_ref, invc_ref, h_ref, wl_ref, b_ref, wr_ref, o_ref):
    blk = ps_ref.shape[1]
    nr = blk // _LANES
    summed = (ps_ref[0] + ps_ref[1]).reshape(nr, _LANES, _LANES)
    mean = (summed * invc_ref[...][:, :, None]).reshape(blk, _LANES)
    o = (jnp.dot(mean, wl_ref[...], preferred_element_type=jnp.float32)
         + b_ref[...]
         + jnp.dot(h_ref[...], wr_ref[...], preferred_element_type=jnp.float32))
    nrm = jnp.sqrt(jnp.sum(o * o, axis=1, keepdims=True))
    o = o / jnp.maximum(nrm, 1e-12)
    o_ref[...] = jnp.maximum(o, 0.0)


def _tc_layer1(p, x, wl, b, wr, n_pad, blk):
    nr = blk // _LANES
    return pl.pallas_call(
        _layer1_block,
        grid=(n_pad // blk,),
        in_specs=[
            pl.BlockSpec((2, blk, _LANES), lambda i: (0, i, 0)),
            pl.BlockSpec((2, nr, _LANES),
                         lambda i: (0, n_pad // nr + i, 0)),
            pl.BlockSpec((blk, _LANES), lambda i: (i, 0)),
            pl.BlockSpec((_LANES, _LANES), lambda i: (0, 0)),
            pl.BlockSpec((1, _LANES), lambda i: (0, 0)),
            pl.BlockSpec((_LANES, _LANES), lambda i: (0, 0)),
        ],
        out_specs=[
            pl.BlockSpec((blk, _LANES), lambda i: (i, 0)),
            pl.BlockSpec((nr, _LANES), lambda i: (i, 0)),
        ],
        out_shape=[
            jax.ShapeDtypeStruct((n_pad, _LANES), jnp.float32),
            jax.ShapeDtypeStruct((n_pad // _LANES, _LANES), jnp.float32),
        ],
    )(p, p, x, wl, b, wr)


def _tc_layer2(p2, invc, h, wl, b, wr, n_out, blk):
    nr = blk // _LANES
    grid = -(-n_out // blk)
    return pl.pallas_call(
        _layer2_block,
        grid=(grid,),
        in_specs=[
            pl.BlockSpec((2, blk, _LANES), lambda i: (0, i, 0)),
            pl.BlockSpec((nr, _LANES), lambda i: (i, 0)),
            pl.BlockSpec((blk, _LANES), lambda i: (i, 0)),
            pl.BlockSpec((_LANES, _LANES), lambda i: (0, 0)),
            pl.BlockSpec((1, _LANES), lambda i: (0, 0)),
            pl.BlockSpec((_LANES, _LANES), lambda i: (0, 0)),
        ],
        out_specs=pl.BlockSpec((blk, _LANES), lambda i: (i, 0)),
        out_shape=jax.ShapeDtypeStruct((n_out, _LANES), jnp.float32),
    )(p2, invc, h, wl, b, wr)


def kernel(matrix_nodes_features, edge_index, W1l, b1, W1r, W2l, b2, W2r):
    x = matrix_nodes_features
    n, d = x.shape
    e = edge_index.shape[1]
    n_pad = -(-n // 2048) * 2048
    blk = 2048

    src = edge_index[0].astype(jnp.int32)
    dst = edge_index[1].astype(jnp.int32)
    ept = e // _NW
    nch = ept // _CH
    src3 = src.reshape(_NW, nch, _CH)
    dst3 = dst.reshape(_NW, nch, _CH)

    x_pad = jnp.zeros((n_pad, d), jnp.float32).at[:n].set(x)
    n_acc1 = n_pad + _LANES
    z_cnt = jnp.zeros((n_acc1 // _NS, _LANES), jnp.float32)
    z_pln = jnp.zeros((n_pad // _NS, _LANES), jnp.float32)

    agg1 = _make_agg(n_pad, e, True)
    agg2 = _make_agg(n_pad, e, False)

    p1 = agg1(x_pad, src3, dst3, z_cnt)
    h, invc = _tc_layer1(p1, x_pad, W1l, b1.reshape(1, -1), W1r, n_pad, blk)
    p2 = agg2(h, src3, dst3, z_pln)
    return _tc_layer2(p2, invc, h, W2l, b2.reshape(1, -1), W2r, n, blk)


# speedup vs baseline: 1.0506x; 1.0158x over previous
import functools

import jax
import jax.numpy as jnp
from jax import lax
from jax.experimental import pallas as pl
from jax.experimental.pallas import tpu as pltpu
from jax.experimental.pallas import tpu_sc as plsc

_NC = 2
_NS = 16
_NW = _NC * _NS
_CH = 80
_LANES = 128


@functools.lru_cache(maxsize=None)
def _make_agg(n_pad: int, n_edges: int, with_counts: bool):
    ept = n_edges // _NW
    nch = ept // _CH
    ncr = n_pad // _LANES
    n_acc = n_pad + (_LANES if with_counts else 0)
    rpt = n_acc // _NS
    mesh = plsc.VectorSubcoreMesh(core_axis_name="c", subcore_axis_name="s")

    assert nch % 2 == 1

    scratch = [
        pltpu.VMEM((_CH,), jnp.int32),
        pltpu.VMEM((_CH,), jnp.int32),
        pltpu.VMEM((nch, _CH), jnp.int32),
        pltpu.VMEM((_CH, _LANES), jnp.float32),
        pltpu.VMEM((_CH, _LANES), jnp.float32),
        pltpu.VMEM_SHARED((n_acc, _LANES), jnp.float32),
        pltpu.SemaphoreType.DMA,
        pltpu.SemaphoreType.DMA,
        pltpu.SemaphoreType.DMA,
        pltpu.SemaphoreType.DMA,
        pltpu.SemaphoreType.DMA,
        pltpu.SemaphoreType.DMA,
    ]
    if with_counts:
        scratch += [
            pltpu.VMEM((ncr, _LANES), jnp.float32),
            pltpu.VMEM((ncr,), jnp.int32),
        ]

    @functools.partial(
        pl.kernel,
        mesh=mesh,
        compiler_params=pltpu.CompilerParams(needs_layout_passes=False),
        out_type=jax.ShapeDtypeStruct((_NC, n_acc, _LANES), jnp.float32),
        scratch_types=scratch,
    )
    def agg(x_hbm, src_hbm, dst_hbm, zeros_hbm, out_hbm,
            srcb_a, srcb_b, dst_v, rows_a, rows_b, acc_sh,
            gsem_a, gsem_b, ssem_a, ssem_b, isem_a, isem_b, *extra):
        c = lax.axis_index("c")
        s = lax.axis_index("s")
        wid = s * _NC + c
        pltpu.sync_copy(zeros_hbm, acc_sh.at[pl.ds(s * rpt, rpt)])
        pltpu.sync_copy(dst_hbm.at[wid], dst_v)
        if with_counts:
            hist_v, cidx_v = extra
            pltpu.sync_copy(zeros_hbm.at[pl.ds(0, ncr)], hist_v)
            for k in range(ncr // 16):
                cidx_v[pl.ds(16 * k, 16)] = (
                    n_pad + 16 * k + lax.iota(jnp.int32, 16))
        plsc.subcore_barrier()

        ones16 = jnp.ones((16,), jnp.float32)
        buf_a = (srcb_a, rows_a, gsem_a, ssem_a, isem_a)
        buf_b = (srcb_b, rows_b, gsem_b, ssem_b, isem_b)

        def hist_add(j):
            if with_counts:
                for k in range(_CH // 16):
                    d16 = dst_v[j, pl.ds(16 * k, 16)]
                    plsc.addupdate_scatter(
                        hist_v,
                        [jnp.right_shift(d16, 7), jnp.bitwise_and(d16, 127)],
                        ones16)

        def handle(j, me, ot):
            srcb_me, rows_me, gsem_me, ssem_me, isem_me = me
            srcb_ot, rows_ot, gsem_ot, ssem_ot, isem_ot = ot

            @pl.when(j > 0)
            def _():
                pltpu.make_async_copy(
                    rows_ot, acc_sh.at[dst_v.at[j - 1]], ssem_ot).wait()

            pltpu.make_async_copy(
                src_hbm.at[wid, j + 1], srcb_ot, isem_ot).wait()
            pltpu.async_copy(x_hbm.at[srcb_ot], rows_ot, gsem_ot)
            hist_add(j)
            pltpu.make_async_copy(x_hbm.at[srcb_me], rows_me, gsem_me).wait()

            @pl.when(j + 2 < nch)
            def _():
                pltpu.async_copy(src_hbm.at[wid, j + 2], srcb_me, isem_me)

            pltpu.async_copy(rows_me, acc_sh.at[dst_v.at[j]], ssem_me,
                             add=True)

        pltpu.sync_copy(src_hbm.at[wid, 0], srcb_a)
        pltpu.async_copy(x_hbm.at[srcb_a], rows_a, gsem_a)
        pltpu.async_copy(src_hbm.at[wid, 1], srcb_b, isem_b)

        def pair(i, carry):
            j0 = 2 * i
            handle(j0, buf_a, buf_b)
            handle(j0 + 1, buf_b, buf_a)
            return carry

        lax.fori_loop(0, (nch - 1) // 2, pair, 0)

        jt = nch - 1
        pltpu.make_async_copy(
            rows_b, acc_sh.at[dst_v.at[jt - 1]], ssem_b).wait()
        pltpu.make_async_copy(x_hbm.at[srcb_a], rows_a, gsem_a).wait()
        pltpu.sync_copy(rows_a, acc_sh.at[dst_v.at[jt]], add=True)
        hist_add(jt)

        if with_counts:
            pltpu.sync_copy(hist_v, acc_sh.at[cidx_v], add=True)
        plsc.subcore_barrier()
        pltpu.sync_copy(acc_sh.at[pl.ds(s * rpt, rpt)],
                        out_hbm.at[c, pl.ds(s * rpt, rpt)])

    return agg


def _layer1_block(ps_ref, pc_ref, x_ref, wl_ref, b_ref, wr_ref,
                  h_ref, invc_ref):
    blk = ps_ref.shape[1]
    nr = blk // _LANES
    cnt = pc_ref[0] + pc_ref[1]
    invc = 1.0 / jnp.maximum(cnt, 1.0)
    summed = (ps_ref[0] + ps_ref[1]).reshape(nr, _LANES, _LANES)
    mean = (summed * invc[:, :, None]).reshape(blk, _LANES)
    h = (jnp.dot(mean, wl_ref[...], preferred_element_type=jnp.float32)
         + b_ref[...]
         + jnp.dot(x_ref[...], wr_ref[...], preferred_element_type=jnp.float32))
    h_ref[...] = jnp.maximum(h, 0.0)
    invc_ref[...] = invc


def _layer2_block(ps_ref, invc_ref, h_ref, wl_ref, b_ref, wr_ref, o_ref):
    blk = ps_ref.shape[1]
    nr = blk // _LANES
    summed = (ps_ref[0] + ps_ref[1]).reshape(nr, _LANES, _LANES)
    mean = (summed * invc_ref[...][:, :, None]).reshape(blk, _LANES)
    o = (jnp.dot(mean, wl_ref[...], preferred_element_type=jnp.float32)
         + b_ref[...]
         + jnp.dot(h_ref[...], wr_ref[...], preferred_element_type=jnp.float32))
    nrm = jnp.sqrt(jnp.sum(o * o, axis=1, keepdims=True))
    o = o / jnp.maximum(nrm, 1e-12)
    o_ref[...] = jnp.maximum(o, 0.0)


def _tc_layer1(p, x, wl, b, wr, n_pad, blk):
    nr = blk // _LANES
    return pl.pallas_call(
        _layer1_block,
        grid=(n_pad // blk,),
        in_specs=[
            pl.BlockSpec((2, blk, _LANES), lambda i: (0, i, 0)),
            pl.BlockSpec((2, nr, _LANES),
                         lambda i: (0, n_pad // nr + i, 0)),
            pl.BlockSpec((blk, _LANES), lambda i: (i, 0)),
            pl.BlockSpec((_LANES, _LANES), lambda i: (0, 0)),
            pl.BlockSpec((1, _LANES), lambda i: (0, 0)),
            pl.BlockSpec((_LANES, _LANES), lambda i: (0, 0)),
        ],
        out_specs=[
            pl.BlockSpec((blk, _LANES), lambda i: (i, 0)),
            pl.BlockSpec((nr, _LANES), lambda i: (i, 0)),
        ],
        out_shape=[
            jax.ShapeDtypeStruct((n_pad, _LANES), jnp.float32),
            jax.ShapeDtypeStruct((n_pad // _LANES, _LANES), jnp.float32),
        ],
    )(p, p, x, wl, b, wr)


def _tc_layer2(p2, invc, h, wl, b, wr, n_out, blk):
    nr = blk // _LANES
    grid = -(-n_out // blk)
    return pl.pallas_call(
        _layer2_block,
        grid=(grid,),
        in_specs=[
            pl.BlockSpec((2, blk, _LANES), lambda i: (0, i, 0)),
            pl.BlockSpec((nr, _LANES), lambda i: (i, 0)),
            pl.BlockSpec((blk, _LANES), lambda i: (i, 0)),
            pl.BlockSpec((_LANES, _LANES), lambda i: (0, 0)),
            pl.BlockSpec((1, _LANES), lambda i: (0, 0)),
            pl.BlockSpec((_LANES, _LANES), lambda i: (0, 0)),
        ],
        out_specs=pl.BlockSpec((blk, _LANES), lambda i: (i, 0)),
        out_shape=jax.ShapeDtypeStruct((n_out, _LANES), jnp.float32),
    )(p2, invc, h, wl, b, wr)


def kernel(matrix_nodes_features, edge_index, W1l, b1, W1r, W2l, b2, W2r):
    x = matrix_nodes_features
    n, d = x.shape
    e = edge_index.shape[1]
    n_pad = -(-n // 2048) * 2048
    blk = 2048

    src = edge_index[0].astype(jnp.int32)
    dst = edge_index[1].astype(jnp.int32)
    ept = e // _NW
    nch = ept // _CH
    src3 = src.reshape(_NW, nch, _CH)
    dst3 = dst.reshape(_NW, nch, _CH)

    n_acc1 = n_pad + _LANES
    z_cnt = jnp.zeros((n_acc1 // _NS, _LANES), jnp.float32)
    z_pln = jnp.zeros((n_pad // _NS, _LANES), jnp.float32)

    agg1 = _make_agg(n_pad, e, True)
    agg2 = _make_agg(n_pad, e, False)

    p1 = agg1(x, src3, dst3, z_cnt)
    h, invc = _tc_layer1(p1, x, W1l, b1.reshape(1, -1), W1r, n_pad, blk)
    p2 = agg2(h, src3, dst3, z_pln)
    return _tc_layer2(p2, invc, h, W2l, b2.reshape(1, -1), W2r, n, blk)


# speedup vs baseline: 1.0628x; 1.0116x over previous
import functools

import jax
import jax.numpy as jnp
from jax import lax
from jax.experimental import pallas as pl
from jax.experimental.pallas import tpu as pltpu
from jax.experimental.pallas import tpu_sc as plsc

_NC = 2
_NS = 16
_NW = _NC * _NS
_CH = 80
_LANES = 128


@functools.lru_cache(maxsize=None)
def _make_agg(n_pad: int, n_edges: int, with_counts: bool):
    ept = n_edges // _NW
    nch = ept // _CH
    ncr = n_pad // _LANES
    n_acc = n_pad + (_LANES if with_counts else 0)
    rpt = n_acc // _NS
    mesh = plsc.VectorSubcoreMesh(core_axis_name="c", subcore_axis_name="s")

    assert nch % 2 == 1

    scratch = [
        pltpu.VMEM((_CH,), jnp.int32),
        pltpu.VMEM((_CH,), jnp.int32),
        pltpu.VMEM((nch, _CH), jnp.int32),
        pltpu.VMEM((_CH, _LANES), jnp.float32),
        pltpu.VMEM((_CH, _LANES), jnp.float32),
        pltpu.VMEM_SHARED((n_acc, _LANES), jnp.float32),
        pltpu.SemaphoreType.DMA,
        pltpu.SemaphoreType.DMA,
        pltpu.SemaphoreType.DMA,
        pltpu.SemaphoreType.DMA,
        pltpu.SemaphoreType.DMA,
        pltpu.SemaphoreType.DMA,
        pltpu.SemaphoreType.DMA,
    ]
    if with_counts:
        scratch += [
            pltpu.VMEM((ncr, _LANES), jnp.float32),
            pltpu.VMEM((ncr,), jnp.int32),
        ]

    @functools.partial(
        pl.kernel,
        mesh=mesh,
        compiler_params=pltpu.CompilerParams(needs_layout_passes=False),
        out_type=jax.ShapeDtypeStruct((_NC, n_acc, _LANES), jnp.float32),
        scratch_types=scratch,
    )
    def agg(x_hbm, src_hbm, dst_hbm, zeros_hbm, out_hbm,
            srcb_a, srcb_b, dst_v, rows_a, rows_b, acc_sh,
            gsem_a, gsem_b, ssem_a, ssem_b, isem_a, isem_b, zsem, *extra):
        c = lax.axis_index("c")
        s = lax.axis_index("s")
        wid = s * _NC + c
        zero = pltpu.async_copy(
            zeros_hbm, acc_sh.at[pl.ds(s * rpt, rpt)], zsem)
        pltpu.sync_copy(dst_hbm.at[wid], dst_v)
        if with_counts:
            hist_v, cidx_v = extra
            pltpu.sync_copy(zeros_hbm.at[pl.ds(0, ncr)], hist_v)
            for k in range(ncr // 16):
                cidx_v[pl.ds(16 * k, 16)] = (
                    n_pad + 16 * k + lax.iota(jnp.int32, 16))

        ones16 = jnp.ones((16,), jnp.float32)
        buf_a = (srcb_a, rows_a, gsem_a, ssem_a, isem_a)
        buf_b = (srcb_b, rows_b, gsem_b, ssem_b, isem_b)

        def hist_add(j):
            if with_counts:
                for k in range(_CH // 16):
                    d16 = dst_v[j, pl.ds(16 * k, 16)]
                    plsc.addupdate_scatter(
                        hist_v,
                        [jnp.right_shift(d16, 7), jnp.bitwise_and(d16, 127)],
                        ones16)

        def handle(j, me, ot):
            srcb_me, rows_me, gsem_me, ssem_me, isem_me = me
            srcb_ot, rows_ot, gsem_ot, ssem_ot, isem_ot = ot

            @pl.when(j > 0)
            def _():
                pltpu.make_async_copy(
                    rows_ot, acc_sh.at[dst_v.at[j - 1]], ssem_ot).wait()

            pltpu.make_async_copy(
                src_hbm.at[wid, j + 1], srcb_ot, isem_ot).wait()
            pltpu.async_copy(x_hbm.at[srcb_ot], rows_ot, gsem_ot)
            hist_add(j)
            pltpu.make_async_copy(x_hbm.at[srcb_me], rows_me, gsem_me).wait()

            @pl.when(j + 2 < nch)
            def _():
                pltpu.async_copy(src_hbm.at[wid, j + 2], srcb_me, isem_me)

            pltpu.async_copy(rows_me, acc_sh.at[dst_v.at[j]], ssem_me,
                             add=True)

        pltpu.sync_copy(src_hbm.at[wid, 0], srcb_a)
        pltpu.async_copy(x_hbm.at[srcb_a], rows_a, gsem_a)
        pltpu.async_copy(src_hbm.at[wid, 1], srcb_b, isem_b)
        zero.wait()
        plsc.subcore_barrier()

        def pair(i, carry):
            j0 = 2 * i
            handle(j0, buf_a, buf_b)
            handle(j0 + 1, buf_b, buf_a)
            return carry

        lax.fori_loop(0, (nch - 1) // 2, pair, 0)

        jt = nch - 1
        pltpu.make_async_copy(
            rows_b, acc_sh.at[dst_v.at[jt - 1]], ssem_b).wait()
        pltpu.make_async_copy(x_hbm.at[srcb_a], rows_a, gsem_a).wait()
        pltpu.sync_copy(rows_a, acc_sh.at[dst_v.at[jt]], add=True)
        hist_add(jt)

        if with_counts:
            pltpu.sync_copy(hist_v, acc_sh.at[cidx_v], add=True)
        plsc.subcore_barrier()
        pltpu.sync_copy(acc_sh.at[pl.ds(s * rpt, rpt)],
                        out_hbm.at[c, pl.ds(s * rpt, rpt)])

    return agg


def _layer1_block(ps_ref, pc_ref, x_ref, wl_ref, b_ref, wr_ref,
                  h_ref, invc_ref):
    blk = ps_ref.shape[1]
    nr = blk // _LANES
    cnt = pc_ref[0] + pc_ref[1]
    invc = 1.0 / jnp.maximum(cnt, 1.0)
    summed = (ps_ref[0] + ps_ref[1]).reshape(nr, _LANES, _LANES)
    mean = (summed * invc[:, :, None]).reshape(blk, _LANES)
    h = (jnp.dot(mean, wl_ref[...], preferred_element_type=jnp.float32)
         + b_ref[...]
         + jnp.dot(x_ref[...], wr_ref[...], preferred_element_type=jnp.float32))
    h_ref[...] = jnp.maximum(h, 0.0)
    invc_ref[...] = invc


def _layer2_block(ps_ref, invc_ref, h_ref, wl_ref, b_ref, wr_ref, o_ref):
    blk = ps_ref.shape[1]
    nr = blk // _LANES
    summed = (ps_ref[0] + ps_ref[1]).reshape(nr, _LANES, _LANES)
    mean = (summed * invc_ref[...][:, :, None]).reshape(blk, _LANES)
    o = (jnp.dot(mean, wl_ref[...], preferred_element_type=jnp.float32)
         + b_ref[...]
         + jnp.dot(h_ref[...], wr_ref[...], preferred_element_type=jnp.float32))
    nrm = jnp.sqrt(jnp.sum(o * o, axis=1, keepdims=True))
    o = o / jnp.maximum(nrm, 1e-12)
    o_ref[...] = jnp.maximum(o, 0.0)


def _tc_layer1(p, x, wl, b, wr, n_pad, blk):
    nr = blk // _LANES
    return pl.pallas_call(
        _layer1_block,
        grid=(n_pad // blk,),
        in_specs=[
            pl.BlockSpec((2, blk, _LANES), lambda i: (0, i, 0)),
            pl.BlockSpec((2, nr, _LANES),
                         lambda i: (0, n_pad // nr + i, 0)),
            pl.BlockSpec((blk, _LANES), lambda i: (i, 0)),
            pl.BlockSpec((_LANES, _LANES), lambda i: (0, 0)),
            pl.BlockSpec((1, _LANES), lambda i: (0, 0)),
            pl.BlockSpec((_LANES, _LANES), lambda i: (0, 0)),
        ],
        out_specs=[
            pl.BlockSpec((blk, _LANES), lambda i: (i, 0)),
            pl.BlockSpec((nr, _LANES), lambda i: (i, 0)),
        ],
        out_shape=[
            jax.ShapeDtypeStruct((n_pad, _LANES), jnp.float32),
            jax.ShapeDtypeStruct((n_pad // _LANES, _LANES), jnp.float32),
        ],
    )(p, p, x, wl, b, wr)


def _tc_layer2(p2, invc, h, wl, b, wr, n_out, blk):
    nr = blk // _LANES
    grid = -(-n_out // blk)
    return pl.pallas_call(
        _layer2_block,
        grid=(grid,),
        in_specs=[
            pl.BlockSpec((2, blk, _LANES), lambda i: (0, i, 0)),
            pl.BlockSpec((nr, _LANES), lambda i: (i, 0)),
            pl.BlockSpec((blk, _LANES), lambda i: (i, 0)),
            pl.BlockSpec((_LANES, _LANES), lambda i: (0, 0)),
            pl.BlockSpec((1, _LANES), lambda i: (0, 0)),
            pl.BlockSpec((_LANES, _LANES), lambda i: (0, 0)),
        ],
        out_specs=pl.BlockSpec((blk, _LANES), lambda i: (i, 0)),
        out_shape=jax.ShapeDtypeStruct((n_out, _LANES), jnp.float32),
    )(p2, invc, h, wl, b, wr)


def kernel(matrix_nodes_features, edge_index, W1l, b1, W1r, W2l, b2, W2r):
    x = matrix_nodes_features
    n, d = x.shape
    e = edge_index.shape[1]
    n_pad = -(-n // 2048) * 2048
    blk = 2048

    src = edge_index[0].astype(jnp.int32)
    dst = edge_index[1].astype(jnp.int32)
    ept = e // _NW
    nch = ept // _CH
    src3 = src.reshape(_NW, nch, _CH)
    dst3 = dst.reshape(_NW, nch, _CH)

    n_acc1 = n_pad + _LANES
    z_cnt = jnp.zeros((n_acc1 // _NS, _LANES), jnp.float32)
    z_pln = jnp.zeros((n_pad // _NS, _LANES), jnp.float32)

    agg1 = _make_agg(n_pad, e, True)
    agg2 = _make_agg(n_pad, e, False)

    p1 = agg1(x, src3, dst3, z_cnt)
    h, invc = _tc_layer1(p1, x, W1l, b1.reshape(1, -1), W1r, n_pad, blk)
    p2 = agg2(h, src3, dst3, z_pln)
    return _tc_layer2(p2, invc, h, W2l, b2.reshape(1, -1), W2r, n, blk)
